# baseline (speedup 1.0000x reference)
import functools

import jax
import jax.numpy as jnp
from jax import lax
from jax.experimental import pallas as pl
from jax.experimental.pallas import tpu as pltpu
from jax.experimental.pallas import tpu_sc as plsc

N_HEADS = 8
OUT_CH = 16
LANES = 16
NC = 2
NS = 16
ROW = 56
SRCW = 80


def _prep_body(x_ref, w_ref, as_ref, at_ref, src_tab_ref, atgt_ref):
    h = jnp.dot(x_ref[...], w_ref[...], preferred_element_type=jnp.float32)
    asrc = jnp.dot(h, as_ref[...], preferred_element_type=jnp.float32)
    atgt_ref[...] = jnp.dot(h, at_ref[...], preferred_element_type=jnp.float32)
    bits = lax.bitcast_convert_type(h, jnp.uint32)
    rne = (bits + jnp.uint32(0x7FFF) +
           ((bits >> jnp.uint32(16)) & jnp.uint32(1)))
    for k in range(4):
        ev = rne[:, (2 * k) * OUT_CH:(2 * k + 1) * OUT_CH] >> jnp.uint32(16)
        od = rne[:, (2 * k + 1) * OUT_CH:(2 * k + 2) * OUT_CH] & jnp.uint32(0xFFFF0000)
        src_tab_ref[:, k * OUT_CH:(k + 1) * OUT_CH] = (
            (ev | od).astype(jnp.int32))
    src_tab_ref[:, 64:64 + LANES] = lax.bitcast_convert_type(asrc, jnp.int32)


def _edge_body(src_hbm, tgt_hbm, stab_hbm, atgt_hbm, zn_hbm, zd_hbm,
               num_out, den_out,
               idx_s, idx_t, hrows, brows, wrows,
               num_acc, den_acc, isem0, isem1, isem2,
               gsem0, gsem1, gsem2, ssem0, ssem1, ssem2):
    n_out = num_out.shape[1]
    n_edges_pad = src_hbm.shape[0]
    cid = lax.axis_index("c")
    sid = lax.axis_index("s")
    worker = sid * NC + cid
    rpw = n_edges_pad // ROW // (NC * NS)
    base = worker * rpw
    isems = (isem0, isem1, isem2)
    gsems = (gsem0, gsem1, gsem2)
    ssems = (ssem0, ssem1, ssem2)

    npad = num_acc.shape[0]
    zrows = (npad // NS) // 8 * 8
    tail = npad - NS * zrows
    pltpu.sync_copy(zn_hbm.at[pl.ds(sid * zrows, zrows)],
                    num_acc.at[pl.ds(sid * zrows, zrows)])
    pltpu.sync_copy(zd_hbm.at[pl.ds(sid * zrows, zrows)],
                    den_acc.at[pl.ds(sid * zrows, zrows)])
    if tail:
        @pl.when(sid == 0)
        def _ztail():
            pltpu.sync_copy(zn_hbm.at[pl.ds(NS * zrows, tail)],
                            num_acc.at[pl.ds(NS * zrows, tail)])
            pltpu.sync_copy(zd_hbm.at[pl.ds(NS * zrows, tail)],
                            den_acc.at[pl.ds(NS * zrows, tail)])
    plsc.subcore_barrier()

    def issue_idx(r, pi):
        qi = lax.rem(r, 6)
        pltpu.async_copy(src_hbm.at[pl.ds((base + r) * ROW, ROW)],
                         idx_s.at[qi], isems[pi])
        pltpu.async_copy(tgt_hbm.at[pl.ds((base + r) * ROW, ROW)],
                         idx_t.at[qi], isems[pi])

    def wait_idx(r, pi):
        qi = lax.rem(r, 6)
        pltpu.make_async_copy(src_hbm.at[pl.ds((base + r) * ROW, ROW)],
                              idx_s.at[qi], isems[pi]).wait()
        pltpu.make_async_copy(tgt_hbm.at[pl.ds((base + r) * ROW, ROW)],
                              idx_t.at[qi], isems[pi]).wait()

    def issue_gathers(r, bi):
        qi = lax.rem(r, 6)
        sem = gsems[bi]
        pltpu.async_copy(atgt_hbm.at[idx_t.at[qi]], brows.at[bi], sem)
        pltpu.async_copy(stab_hbm.at[idx_s.at[qi]], hrows.at[bi], sem)

    def wait_gathers(r, bi):
        qi = lax.rem(r, 6)
        sem = gsems[bi]
        pltpu.make_async_copy(atgt_hbm.at[idx_t.at[qi]], brows.at[bi], sem).wait()
        pltpu.make_async_copy(stab_hbm.at[idx_s.at[qi]], hrows.at[bi], sem).wait()

    def issue_scatters(r, bi):
        qi = lax.rem(r, 6)
        sem = ssems[bi]
        pltpu.async_copy(brows.at[bi], den_acc.at[idx_t.at[qi]], sem, add=True)
        pltpu.async_copy(wrows.at[bi], num_acc.at[idx_t.at[qi]], sem, add=True)

    def wait_scatters(r, bi):
        qi = lax.rem(r, 6)
        sem = ssems[bi]
        pltpu.make_async_copy(brows.at[bi], den_acc.at[idx_t.at[qi]], sem).wait()
        pltpu.make_async_copy(wrows.at[bi], num_acc.at[idx_t.at[qi]], sem).wait()

    def compute_row(bi):
        @plsc.parallel_loop(0, ROW, unroll=4)
        def _edge(e):
            sa = lax.bitcast_convert_type(hrows[bi, e, pl.ds(64, LANES)],
                                          jnp.float32)
            s = sa + brows[bi, e, :]
            s = jnp.where(s < 0.0, s * jnp.float32(0.01), s)
            p = jnp.exp(s)
            brows[bi, e, :] = p
            for k in range(4):
                v = hrows[bi, e, pl.ds(k * LANES, LANES)]
                lo = lax.bitcast_convert_type(
                    lax.shift_left(v, jnp.int32(16)), jnp.float32)
                hi = lax.bitcast_convert_type(
                    lax.bitwise_and(v, jnp.int32(-65536)), jnp.float32)
                wrows[bi, e, pl.ds((2 * k) * OUT_CH, OUT_CH)] = lo * p[2 * k]
                wrows[bi, e, pl.ds((2 * k + 1) * OUT_CH, OUT_CH)] = (
                    hi * p[2 * k + 1])

    def body(r, bi, first=False, gnext=True, inext=True):
        bn = (bi + 1) % 3
        wait_gathers(r, bi)
        if not first:
            wait_scatters(r - 2, bn)
        if gnext:
            wait_idx(r + 1, bn)
            issue_gathers(r + 1, bn)
        if inext:
            issue_idx(r + 4, bn)
        compute_row(bi)
        issue_scatters(r, bi)

    issue_idx(0, 0)
    issue_idx(1, 1)
    issue_idx(2, 2)
    issue_idx(3, 0)
    wait_idx(0, 0)
    issue_gathers(0, 0)
    body(0, 0, first=True)
    body(1, 1, first=True)
    body(2, 2)

    @pl.loop(0, rpw // 3 - 3)
    def _triple(i):
        for b in range(3):
            body(3 * i + 3 + b, b)

    for rr in range(rpw - 6, rpw):
        body(rr, rr % 3, gnext=(rr < rpw - 1), inext=(rr + 4 < rpw))
    wait_scatters(rpw - 2, (rpw - 2) % 3)
    wait_scatters(rpw - 1, (rpw - 1) % 3)

    plsc.subcore_barrier()
    orows = (n_out // NS) // 8 * 8
    otail = n_out - NS * orows
    pltpu.sync_copy(num_acc.at[pl.ds(sid * orows, orows)],
                    num_out.at[cid].at[pl.ds(sid * orows, orows)])
    pltpu.sync_copy(den_acc.at[pl.ds(sid * orows, orows)],
                    den_out.at[cid].at[pl.ds(sid * orows, orows)])
    if otail:
        @pl.when(sid == 1 % NS)
        def _otail():
            pltpu.sync_copy(num_acc.at[pl.ds(NS * orows, otail)],
                            num_out.at[cid].at[pl.ds(NS * orows, otail)])
            pltpu.sync_copy(den_acc.at[pl.ds(NS * orows, otail)],
                            den_out.at[cid].at[pl.ds(NS * orows, otail)])


def _combine_body(num_ref, den_ref, out_ref):
    num = num_ref[0] + num_ref[1]
    den = den_ref[0] + den_ref[1]
    recip = 1.0 / (den + jnp.float32(1e-16))
    acc = num[:, 0:OUT_CH] * recip[:, 0:1]
    for hh in range(1, N_HEADS):
        acc = acc + num[:, hh * OUT_CH:(hh + 1) * OUT_CH] * recip[:, hh:hh + 1]
    out_ref[...] = acc * jnp.float32(1.0 / N_HEADS)


def kernel(x, edge_indices, W, a_target, a_source):
    n_nodes, d_feat = x.shape
    n_edges = edge_indices.shape[1]

    quantum = NC * NS * 2 * ROW
    e_pad = -(-n_edges // quantum) * quantum
    n_pad = n_nodes + 8
    pad = jnp.full((2, e_pad - n_edges), n_nodes, jnp.int32)
    ei = jnp.concatenate([edge_indices.astype(jnp.int32), pad], axis=1)
    n_rows = e_pad // ROW
    rpw = n_rows // (NC * NS)

    src = ei[0]
    tgt = ei[1]

    hc = N_HEADS * OUT_CH
    k_idx = jnp.arange(hc)
    l_idx = jnp.arange(LANES)
    mask = (k_idx[:, None] // OUT_CH) == (l_idx[None, :] % N_HEADS)
    As = jnp.where(mask, a_source.reshape(-1)[:, None], 0.0).astype(jnp.float32)
    At = jnp.where(mask, a_target.reshape(-1)[:, None], 0.0).astype(jnp.float32)

    src_tab, atgt = pl.pallas_call(
        _prep_body,
        out_shape=[
            jax.ShapeDtypeStruct((n_nodes, SRCW), jnp.int32),
            jax.ShapeDtypeStruct((n_nodes, LANES), jnp.float32),
        ],
    )(x, W, As, At)

    pad_rows = n_pad - n_nodes
    src_tab = jnp.pad(src_tab, ((0, pad_rows), (0, 0)))
    atgt = jnp.pad(atgt, ((0, pad_rows), (0, 0)))
    zeros_num = jnp.zeros((n_pad, hc), jnp.float32)
    zeros_den = jnp.zeros((n_pad, LANES), jnp.float32)

    mesh = plsc.VectorSubcoreMesh(core_axis_name="c", subcore_axis_name="s",
                                  num_cores=NC, num_subcores=NS)
    edge_kernel = pl.kernel(
        _edge_body,
        out_type=[
            jax.ShapeDtypeStruct((NC, n_nodes, hc), jnp.float32),
            jax.ShapeDtypeStruct((NC, n_nodes, LANES), jnp.float32),
        ],
        mesh=mesh,
        compiler_params=pltpu.CompilerParams(use_tc_tiling_on_sc=False),
        scratch_types=[
            pltpu.VMEM((6, ROW), jnp.int32),
            pltpu.VMEM((6, ROW), jnp.int32),
            pltpu.VMEM((3, ROW, SRCW), jnp.int32),
            pltpu.VMEM((3, ROW, LANES), jnp.float32),
            pltpu.VMEM((3, ROW, hc), jnp.float32),
            pltpu.VMEM_SHARED((n_pad, hc), jnp.float32),
            pltpu.VMEM_SHARED((n_pad, LANES), jnp.float32),
            pltpu.SemaphoreType.DMA,
            pltpu.SemaphoreType.DMA,
            pltpu.SemaphoreType.DMA,
            pltpu.SemaphoreType.DMA,
            pltpu.SemaphoreType.DMA,
            pltpu.SemaphoreType.DMA,
            pltpu.SemaphoreType.DMA,
            pltpu.SemaphoreType.DMA,
            pltpu.SemaphoreType.DMA,
        ],
    )
    num_p, den_p = edge_kernel(src, tgt, src_tab, atgt,
                               zeros_num, zeros_den)

    bn = 2000
    out = pl.pallas_call(
        _combine_body,
        grid=(n_nodes // bn,),
        in_specs=[
            pl.BlockSpec((NC, bn, hc), lambda i: (0, i, 0)),
            pl.BlockSpec((NC, bn, LANES), lambda i: (0, i, 0)),
        ],
        out_specs=pl.BlockSpec((bn, OUT_CH), lambda i: (i, 0)),
        out_shape=jax.ShapeDtypeStruct((n_nodes, OUT_CH), jnp.float32),
    )(num_p, den_p)
    return out


# speedup vs baseline: 1.3620x; 1.3620x over previous
import functools

import jax
import jax.numpy as jnp
from jax import lax
from jax.experimental import pallas as pl
from jax.experimental.pallas import tpu as pltpu
from jax.experimental.pallas import tpu_sc as plsc

N_HEADS = 8
OUT_CH = 16
LANES = 16
NC = 2
NS = 16
ROW = 88
SRCW = 80


def _prep_body(x_ref, w_ref, as_ref, at_ref, src_tab_ref, atgt_ref):
    h = jnp.dot(x_ref[...], w_ref[...], preferred_element_type=jnp.float32)
    asrc = jnp.dot(h, as_ref[...], preferred_element_type=jnp.float32)
    atgt_ref[...] = jnp.dot(h, at_ref[...], preferred_element_type=jnp.float32)
    bits = lax.bitcast_convert_type(h, jnp.uint32)
    rne = (bits + jnp.uint32(0x7FFF) +
           ((bits >> jnp.uint32(16)) & jnp.uint32(1)))
    for k in range(4):
        ev = rne[:, (2 * k) * OUT_CH:(2 * k + 1) * OUT_CH] >> jnp.uint32(16)
        od = rne[:, (2 * k + 1) * OUT_CH:(2 * k + 2) * OUT_CH] & jnp.uint32(0xFFFF0000)
        src_tab_ref[:, k * OUT_CH:(k + 1) * OUT_CH] = (
            (ev | od).astype(jnp.int32))
    src_tab_ref[:, 64:64 + LANES] = lax.bitcast_convert_type(asrc, jnp.int32)


def _edge_body(src_hbm, tgt_hbm, stab_hbm, atgt_hbm, zn_hbm, zd_hbm,
               num_out, den_out,
               idx_s, idx_t, hrows, brows, wrows,
               num_acc, den_acc, isem0, isem1, gsem0, gsem1, ssem0, ssem1):
    n_out = num_out.shape[1]
    n_edges_pad = src_hbm.shape[0]
    cid = lax.axis_index("c")
    sid = lax.axis_index("s")
    worker = sid * NC + cid
    rpw = n_edges_pad // ROW // (NC * NS)
    base = worker * rpw
    isems = (isem0, isem1)
    gsems = (gsem0, gsem1)
    ssems = (ssem0, ssem1)


    def issue_idx(r, pi):
        qi = lax.rem(r, 4)
        pltpu.async_copy(src_hbm.at[pl.ds((base + r) * ROW, ROW)],
                         idx_s.at[qi], isems[pi])
        pltpu.async_copy(tgt_hbm.at[pl.ds((base + r) * ROW, ROW)],
                         idx_t.at[qi], isems[pi])

    def wait_idx(r, pi):
        qi = lax.rem(r, 4)
        pltpu.make_async_copy(src_hbm.at[pl.ds((base + r) * ROW, ROW)],
                              idx_s.at[qi], isems[pi]).wait()
        pltpu.make_async_copy(tgt_hbm.at[pl.ds((base + r) * ROW, ROW)],
                              idx_t.at[qi], isems[pi]).wait()

    def issue_gathers(r, bi):
        qi = lax.rem(r, 4)
        sem = gsems[bi]
        pltpu.async_copy(atgt_hbm.at[idx_t.at[qi]], brows.at[bi], sem)
        pltpu.async_copy(stab_hbm.at[idx_s.at[qi]], hrows.at[bi], sem)

    def wait_gathers(r, bi):
        qi = lax.rem(r, 4)
        sem = gsems[bi]
        pltpu.make_async_copy(atgt_hbm.at[idx_t.at[qi]], brows.at[bi], sem).wait()
        pltpu.make_async_copy(stab_hbm.at[idx_s.at[qi]], hrows.at[bi], sem).wait()

    def issue_scatters(r, bi):
        qi = lax.rem(r, 4)
        sem = ssems[bi]
        pltpu.async_copy(brows.at[bi], den_acc.at[idx_t.at[qi]], sem, add=True)
        pltpu.async_copy(wrows.at[bi], num_acc.at[idx_t.at[qi]], sem, add=True)

    def wait_scatters(r, bi):
        qi = lax.rem(r, 4)
        sem = ssems[bi]
        pltpu.make_async_copy(brows.at[bi], den_acc.at[idx_t.at[qi]], sem).wait()
        pltpu.make_async_copy(wrows.at[bi], num_acc.at[idx_t.at[qi]], sem).wait()

    def compute_row(bi):
        @plsc.parallel_loop(0, ROW, unroll=4)
        def _edge(e):
            sa = lax.bitcast_convert_type(hrows[bi, e, pl.ds(64, LANES)],
                                          jnp.float32)
            s = sa + brows[bi, e, :]
            s = jnp.where(s < 0.0, s * jnp.float32(0.01), s)
            p = jnp.exp(s)
            brows[bi, e, :] = p
            for k in range(4):
                v = hrows[bi, e, pl.ds(k * LANES, LANES)]
                lo = lax.bitcast_convert_type(
                    lax.shift_left(v, jnp.int32(16)), jnp.float32)
                hi = lax.bitcast_convert_type(
                    lax.bitwise_and(v, jnp.int32(-65536)), jnp.float32)
                wrows[bi, e, pl.ds((2 * k) * OUT_CH, OUT_CH)] = lo * p[2 * k]
                wrows[bi, e, pl.ds((2 * k + 1) * OUT_CH, OUT_CH)] = (
                    hi * p[2 * k + 1])

    def body(r, bi, first=False, gnext=True, inext=True):
        wait_gathers(r, bi)
        if not first:
            wait_scatters(r - 1, bi ^ 1)
        if gnext:
            wait_idx(r + 1, bi ^ 1)
            issue_gathers(r + 1, bi ^ 1)
        if inext:
            issue_idx(r + 3, bi ^ 1)
        compute_row(bi)
        issue_scatters(r, bi)

    issue_idx(0, 0)
    issue_idx(1, 1)
    issue_idx(2, 0)
    wait_idx(0, 0)
    issue_gathers(0, 0)
    npad = num_acc.shape[0]
    zrows = (npad // NS) // 8 * 8
    tail = npad - NS * zrows
    pltpu.sync_copy(zn_hbm.at[pl.ds(sid * zrows, zrows)],
                    num_acc.at[pl.ds(sid * zrows, zrows)])
    pltpu.sync_copy(zd_hbm.at[pl.ds(sid * zrows, zrows)],
                    den_acc.at[pl.ds(sid * zrows, zrows)])
    if tail:
        @pl.when(sid == 0)
        def _ztail():
            pltpu.sync_copy(zn_hbm.at[pl.ds(NS * zrows, tail)],
                            num_acc.at[pl.ds(NS * zrows, tail)])
            pltpu.sync_copy(zd_hbm.at[pl.ds(NS * zrows, tail)],
                            den_acc.at[pl.ds(NS * zrows, tail)])
    plsc.subcore_barrier()
    body(0, 0, first=True)

    @pl.loop(0, (rpw - 4) // 2)
    def _pair(i):
        for b in range(2):
            body(2 * i + 1 + b, 1 - b)

    body(rpw - 3, (rpw - 3) % 2, inext=False)
    body(rpw - 2, (rpw - 2) % 2, inext=False)
    body(rpw - 1, (rpw - 1) % 2, gnext=False, inext=False)
    wait_scatters(rpw - 1, (rpw - 1) % 2)

    plsc.subcore_barrier()
    orows = (n_out // NS) // 8 * 8
    otail = n_out - NS * orows
    pltpu.sync_copy(num_acc.at[pl.ds(sid * orows, orows)],
                    num_out.at[cid].at[pl.ds(sid * orows, orows)])
    pltpu.sync_copy(den_acc.at[pl.ds(sid * orows, orows)],
                    den_out.at[cid].at[pl.ds(sid * orows, orows)])
    if otail:
        @pl.when(sid == 1 % NS)
        def _otail():
            pltpu.sync_copy(num_acc.at[pl.ds(NS * orows, otail)],
                            num_out.at[cid].at[pl.ds(NS * orows, otail)])
            pltpu.sync_copy(den_acc.at[pl.ds(NS * orows, otail)],
                            den_out.at[cid].at[pl.ds(NS * orows, otail)])


def _combine_body(num_ref, den_ref, out_ref):
    num = num_ref[0] + num_ref[1]
    den = den_ref[0] + den_ref[1]
    recip = 1.0 / (den + jnp.float32(1e-16))
    acc = num[:, 0:OUT_CH] * recip[:, 0:1]
    for hh in range(1, N_HEADS):
        acc = acc + num[:, hh * OUT_CH:(hh + 1) * OUT_CH] * recip[:, hh:hh + 1]
    out_ref[...] = acc * jnp.float32(1.0 / N_HEADS)


def kernel(x, edge_indices, W, a_target, a_source):
    n_nodes, d_feat = x.shape
    n_edges = edge_indices.shape[1]

    quantum = NC * NS * 2 * ROW
    e_pad = -(-n_edges // quantum) * quantum
    n_pad = n_nodes + 8
    pad = jnp.full((2, e_pad - n_edges), n_nodes, jnp.int32)
    ei = jnp.concatenate([edge_indices.astype(jnp.int32), pad], axis=1)
    n_rows = e_pad // ROW
    rpw = n_rows // (NC * NS)

    src = ei[0]
    tgt = ei[1]

    hc = N_HEADS * OUT_CH
    k_idx = jnp.arange(hc)
    l_idx = jnp.arange(LANES)
    mask = (k_idx[:, None] // OUT_CH) == (l_idx[None, :] % N_HEADS)
    As = jnp.where(mask, a_source.reshape(-1)[:, None], 0.0).astype(jnp.float32)
    At = jnp.where(mask, a_target.reshape(-1)[:, None], 0.0).astype(jnp.float32)

    src_tab, atgt = pl.pallas_call(
        _prep_body,
        out_shape=[
            jax.ShapeDtypeStruct((n_nodes, SRCW), jnp.int32),
            jax.ShapeDtypeStruct((n_nodes, LANES), jnp.float32),
        ],
    )(x, W, As, At)

    pad_rows = n_pad - n_nodes
    src_tab = jnp.pad(src_tab, ((0, pad_rows), (0, 0)))
    atgt = jnp.pad(atgt, ((0, pad_rows), (0, 0)))
    zeros_num = jnp.zeros((n_pad, hc), jnp.float32)
    zeros_den = jnp.zeros((n_pad, LANES), jnp.float32)

    mesh = plsc.VectorSubcoreMesh(core_axis_name="c", subcore_axis_name="s",
                                  num_cores=NC, num_subcores=NS)
    edge_kernel = pl.kernel(
        _edge_body,
        out_type=[
            jax.ShapeDtypeStruct((NC, n_nodes, hc), jnp.float32),
            jax.ShapeDtypeStruct((NC, n_nodes, LANES), jnp.float32),
        ],
        mesh=mesh,
        compiler_params=pltpu.CompilerParams(use_tc_tiling_on_sc=False),
        scratch_types=[
            pltpu.VMEM((4, ROW), jnp.int32),
            pltpu.VMEM((4, ROW), jnp.int32),
            pltpu.VMEM((2, ROW, SRCW), jnp.int32),
            pltpu.VMEM((2, ROW, LANES), jnp.float32),
            pltpu.VMEM((2, ROW, hc), jnp.float32),
            pltpu.VMEM_SHARED((n_pad, hc), jnp.float32),
            pltpu.VMEM_SHARED((n_pad, LANES), jnp.float32),
            pltpu.SemaphoreType.DMA,
            pltpu.SemaphoreType.DMA,
            pltpu.SemaphoreType.DMA,
            pltpu.SemaphoreType.DMA,
            pltpu.SemaphoreType.DMA,
            pltpu.SemaphoreType.DMA,
        ],
    )
    num_p, den_p = edge_kernel(src, tgt, src_tab, atgt,
                               zeros_num, zeros_den)

    bn = 2000
    out = pl.pallas_call(
        _combine_body,
        grid=(n_nodes // bn,),
        in_specs=[
            pl.BlockSpec((NC, bn, hc), lambda i: (0, i, 0)),
            pl.BlockSpec((NC, bn, LANES), lambda i: (0, i, 0)),
        ],
        out_specs=pl.BlockSpec((bn, OUT_CH), lambda i: (i, 0)),
        out_shape=jax.ShapeDtypeStruct((n_nodes, OUT_CH), jnp.float32),
    )(num_p, den_p)
    return out
